# baseline (speedup 1.0000x reference)
import functools

import jax
import jax.numpy as jnp
from jax import lax
from jax.experimental import pallas as pl
from jax.experimental.pallas import tpu as pltpu
from jax.experimental.pallas import tpu_sc as plsc

_NUM_CORES = 2
_NUM_SUBCORES = 16
_NUM_WORKERS = _NUM_CORES * _NUM_SUBCORES


@functools.partial(jax.jit, static_argnames=())
def _gather(data, idx):
    B = idx.shape[0]
    D = data.shape[1]
    b_per_w = B // _NUM_WORKERS

    mesh = plsc.VectorSubcoreMesh(core_axis_name="c", subcore_axis_name="s")

    @functools.partial(
        pl.kernel,
        mesh=mesh,
        out_type=jax.ShapeDtypeStruct((B, D), jnp.float32),
        scratch_types=[
            pltpu.VMEM((b_per_w,), jnp.int32),
            pltpu.VMEM((b_per_w, D), jnp.float32),
            pltpu.SemaphoreType.DMA,
        ],
        compiler_params=pltpu.CompilerParams(use_tc_tiling_on_sc=False),
    )
    def k(table_hbm, idx_hbm, out_hbm, idx_v, rows_v, sem):
        wid = lax.axis_index("s") * _NUM_CORES + lax.axis_index("c")
        base = wid * b_per_w
        pltpu.sync_copy(idx_hbm.at[pl.ds(base, b_per_w)], idx_v)
        pltpu.async_copy(table_hbm.at[idx_v], rows_v, sem).wait()
        pltpu.sync_copy(rows_v, out_hbm.at[pl.ds(base, b_per_w)])

    return k(data, idx)


def kernel(data, channelindex):
    return _gather(data, channelindex.astype(jnp.int32))


# speedup vs baseline: 1.2808x; 1.2808x over previous
import functools

import jax
import jax.numpy as jnp
from jax import lax
from jax.experimental import pallas as pl
from jax.experimental.pallas import tpu as pltpu
from jax.experimental.pallas import tpu_sc as plsc

_NUM_CORES = 2
_NUM_SUBCORES = 16
_NUM_WORKERS = _NUM_CORES * _NUM_SUBCORES
_L = 16

_V = 100000
_D = 64
_B = 4096
_RANGE = 1536
_NBUCK = (_V + _RANGE - 1) // _RANGE
_WS_LAST = 782 * 128 - _RANGE
_FLUSH = 48
_ICHUNK = 1024


def _splat_lane(vec, r):
    return lax.gather(
        vec,
        jnp.full((_L, 1), r, jnp.int32),
        dimension_numbers=lax.GatherDimensionNumbers(
            offset_dims=(), collapsed_slice_dims=(0,), start_index_map=(0,)),
        slice_sizes=(1,),
        mode=lax.GatherScatterMode.PROMISE_IN_BOUNDS)


def _impl(dataT, idx):
    mesh = plsc.VectorSubcoreMesh(core_axis_name="c", subcore_axis_name="s")

    @functools.partial(
        pl.kernel,
        mesh=mesh,
        out_type=jax.ShapeDtypeStruct((_B, 2 * _D), jnp.float32),
        scratch_types=[
            pltpu.VMEM((_ICHUNK,), jnp.int32),
            pltpu.VMEM((_B + _FLUSH,), jnp.int32),
            pltpu.VMEM((_B + _FLUSH,), jnp.int32),
            pltpu.VMEM((_D, _RANGE), jnp.float32),
            pltpu.VMEM((_FLUSH, 2 * _D), jnp.float32),
            pltpu.VMEM((_B // _FLUSH + 1, _FLUSH), jnp.int32),
            pltpu.SemaphoreType.DMA,
            pltpu.SemaphoreType.DMA,
        ],
        compiler_params=pltpu.CompilerParams(needs_layout_passes=False),
    )
    def k(tab_hbm, idx_hbm, scr_hbm, idxc_v, nbuf, jbuf, win_v, src_v,
          jrow_v, sem, wsem):
        wid = lax.axis_index("s") * _NUM_CORES + lax.axis_index("c")
        iota = lax.iota(jnp.int32, _L)

        for p in range(-(-_NBUCK // _NUM_WORKERS)):
            b = wid + p * _NUM_WORKERS
            lo = b * _RANGE
            hi = lo + _RANGE
            ws = pl.multiple_of(jnp.minimum(lo, _WS_LAST), 128)

            wcp = pltpu.async_copy(
                tab_hbm.at[:, pl.ds(ws, _RANGE)], win_v, wsem)

            cnt = jnp.int32(0)
            for c in range(_B // _ICHUNK):
                pltpu.sync_copy(
                    idx_hbm.at[pl.ds(c * _ICHUNK, _ICHUNK)], idxc_v)

                def scan_body(g, cnt, c=c):
                    v = idxc_v[pl.ds(g * _L, _L)]
                    jv = c * _ICHUNK + g * _L + iota
                    m = jnp.logical_and(v >= lo, v < hi)
                    plsc.store_compressed(nbuf.at[pl.ds(cnt, _L)], v, mask=m)
                    plsc.store_compressed(jbuf.at[pl.ds(cnt, _L)], jv, mask=m)
                    return cnt + jnp.sum(m.astype(jnp.int32))

                cnt = lax.fori_loop(0, _ICHUNK // _L, scan_body, cnt)

            for t in range(_FLUSH // _L):
                jbuf[pl.ds(cnt + t * _L, _L)] = jnp.full((_L,), -1, jnp.int32)
                nbuf[pl.ds(cnt + t * _L, _L)] = jnp.broadcast_to(ws, (_L,))

            wcp.wait()

            def flush_body(f, carry):
                for t in range(_FLUSH // _L):
                    nseg = nbuf[pl.ds(f * _FLUSH + t * _L, _L)] - ws
                    jseg = jbuf[pl.ds(f * _FLUSH + t * _L, _L)]
                    jrow_v[f, pl.ds(t * _L, _L)] = jseg
                    for r in range(_L):
                        nn = _splat_lane(nseg, r)
                        for kk in range(_D // _L):
                            src_v[t * _L + r, pl.ds(kk * _L, _L)] = (
                                plsc.load_gather(
                                    win_v, [iota + kk * _L, nn]))
                pltpu.async_copy(
                    src_v,
                    scr_hbm.at[plsc.Indices(jrow_v.at[f], ignored_value=-1)],
                    sem).wait()
                return carry

            nflush = (cnt + _FLUSH - 1) // _FLUSH
            lax.fori_loop(0, nflush, flush_body, jnp.int32(0))

    return k(dataT, idx)


def kernel(data, channelindex):
    dataT = data.T
    idx = channelindex.astype(jnp.int32)
    scr = _impl(dataT, idx)
    return scr[:, :_D]


# speedup vs baseline: 1.6955x; 1.3238x over previous
import functools

import jax
import jax.numpy as jnp
from jax import lax
from jax.experimental import pallas as pl
from jax.experimental.pallas import tpu as pltpu
from jax.experimental.pallas import tpu_sc as plsc

_NUM_CORES = 2
_NUM_SUBCORES = 16
_NUM_WORKERS = _NUM_CORES * _NUM_SUBCORES
_L = 16

_V = 100000
_D = 64
_B = 4096
_RANGE = 640
_NBUCK = (_V + _RANGE - 1) // _RANGE
_NPASS = -(-_NBUCK // _NUM_WORKERS)
_WS_LAST = 782 * 128 - _RANGE
_MAGIC = 52429
_PAD_N = 1 << 17
_FLUSH = 48
_ICHUNK = 2048


def _splat_lane(vec, r):
    return lax.gather(
        vec,
        jnp.full((_L, 1), r, jnp.int32),
        dimension_numbers=lax.GatherDimensionNumbers(
            offset_dims=(), collapsed_slice_dims=(0,), start_index_map=(0,)),
        slice_sizes=(1,),
        mode=lax.GatherScatterMode.PROMISE_IN_BOUNDS)


def _bucket_of(v):
    return lax.shift_right_logical(
        lax.shift_right_logical(v, 7) * _MAGIC, 18)


def _impl(dataT, idx):
    mesh = plsc.VectorSubcoreMesh(core_axis_name="c", subcore_axis_name="s")

    @functools.partial(
        pl.kernel,
        mesh=mesh,
        out_type=jax.ShapeDtypeStruct((_B, 2 * _D), jnp.float32),
        scratch_types=[
            pltpu.VMEM((_ICHUNK,), jnp.int32),
            pltpu.VMEM((_B + _L,), jnp.int32),
            pltpu.VMEM((_B + _L,), jnp.int32),
            pltpu.VMEM((_B + _FLUSH,), jnp.int32),
            pltpu.VMEM((_B + _FLUSH,), jnp.int32),
            pltpu.VMEM((_D, _RANGE), jnp.float32),
            pltpu.VMEM((_D, _RANGE), jnp.float32),
            pltpu.VMEM((_FLUSH, 2 * _D), jnp.float32),
            pltpu.VMEM((_B // _FLUSH + 1, _FLUSH), jnp.int32),
            pltpu.SemaphoreType.DMA,
            pltpu.SemaphoreType.DMA,
            pltpu.SemaphoreType.DMA,
        ],
        compiler_params=pltpu.CompilerParams(needs_layout_passes=False),
    )
    def k(tab_hbm, idx_hbm, scr_hbm, idxc_v, nA, jA, nB, jB, win0, win1,
          src_v, jrow_v, sem, wsem0, wsem1):
        wid = lax.axis_index("s") * _NUM_CORES + lax.axis_index("c")
        iota = lax.iota(jnp.int32, _L)
        wins = (win0, win1)
        wsems = (wsem0, wsem1)

        def wstart(p):
            lo = (wid + p * _NUM_WORKERS) * _RANGE
            ws = pl.multiple_of(jnp.minimum(lo, _WS_LAST), 128)
            return pltpu.async_copy(
                tab_hbm.at[:, pl.ds(ws, _RANGE)], wins[p % 2], wsems[p % 2])

        wcp = {0: wstart(0)}

        cntA = jnp.int32(0)
        for c in range(_B // _ICHUNK):
            pltpu.sync_copy(idx_hbm.at[pl.ds(c * _ICHUNK, _ICHUNK)], idxc_v)

            def scan_body(g, cnt, c=c):
                v = idxc_v[pl.ds(g * _L, _L)]
                jv = c * _ICHUNK + g * _L + iota
                m = (_bucket_of(v) & (_NUM_WORKERS - 1)) == wid
                plsc.store_compressed(nA.at[pl.ds(cnt, _L)], v, mask=m)
                plsc.store_compressed(jA.at[pl.ds(cnt, _L)], jv, mask=m)
                return cnt + jnp.sum(m.astype(jnp.int32))

            cntA = lax.fori_loop(0, _ICHUNK // _L, scan_body, cntA)
        nA[pl.ds(cntA, _L)] = jnp.full((_L,), _PAD_N, jnp.int32)

        for p in range(_NPASS):
            if p + 1 < _NPASS:
                wcp[p + 1] = wstart(p + 1)

            lo = (wid + p * _NUM_WORKERS) * _RANGE
            ws = jnp.minimum(lo, _WS_LAST)
            win = wins[p % 2]

            def sel_body(g, cnt, p=p):
                v = nA[pl.ds(g * _L, _L)]
                jv = jA[pl.ds(g * _L, _L)]
                m = lax.shift_right_logical(_bucket_of(v), 5) == p
                plsc.store_compressed(nB.at[pl.ds(cnt, _L)], v, mask=m)
                plsc.store_compressed(jB.at[pl.ds(cnt, _L)], jv, mask=m)
                return cnt + jnp.sum(m.astype(jnp.int32))

            ngrp = (cntA + _L) // _L
            cnt = lax.fori_loop(0, ngrp, sel_body, jnp.int32(0))

            for t in range(_FLUSH // _L):
                jB[pl.ds(cnt + t * _L, _L)] = jnp.full((_L,), -1, jnp.int32)
                nB[pl.ds(cnt + t * _L, _L)] = jnp.broadcast_to(ws, (_L,))

            wcp[p].wait()

            def flush_body(f, carry, p=p, win=win, ws=ws):
                for t in range(_FLUSH // _L):
                    nseg = nB[pl.ds(f * _FLUSH + t * _L, _L)] - ws
                    jseg = jB[pl.ds(f * _FLUSH + t * _L, _L)]
                    jrow_v[f, pl.ds(t * _L, _L)] = jseg
                    for r in range(_L):
                        nn = _splat_lane(nseg, r)
                        for kk in range(_D // _L):
                            src_v[t * _L + r, pl.ds(kk * _L, _L)] = (
                                plsc.load_gather(win, [iota + kk * _L, nn]))
                pltpu.async_copy(
                    src_v,
                    scr_hbm.at[plsc.Indices(jrow_v.at[f], ignored_value=-1)],
                    sem).wait()
                return carry

            nflush = (cnt + _FLUSH - 1) // _FLUSH
            lax.fori_loop(0, nflush, flush_body, jnp.int32(0))

    return k(dataT, idx)


def kernel(data, channelindex):
    dataT = data.T
    idx = channelindex.astype(jnp.int32)
    scr = _impl(dataT, idx)
    return scr[:, :_D]


# speedup vs baseline: 1.9044x; 1.1232x over previous
import functools

import jax
import jax.numpy as jnp
from jax import lax
from jax.experimental import pallas as pl
from jax.experimental.pallas import tpu as pltpu
from jax.experimental.pallas import tpu_sc as plsc

_NUM_CORES = 2
_NUM_SUBCORES = 16
_NUM_WORKERS = _NUM_CORES * _NUM_SUBCORES
_L = 16

_V = 100000
_D = 64
_B = 4096
_RANGE = 640
_NBUCK = (_V + _RANGE - 1) // _RANGE
_NPASS = -(-_NBUCK // _NUM_WORKERS)
_WS_LAST = 782 * 128 - _RANGE
_MAGIC = 52429
_PAD_N = 1 << 17
_FLUSH = 32
_ICHUNK = 2048


def _splat_lane(vec, r):
    return lax.gather(
        vec,
        jnp.full((_L, 1), r, jnp.int32),
        dimension_numbers=lax.GatherDimensionNumbers(
            offset_dims=(), collapsed_slice_dims=(0,), start_index_map=(0,)),
        slice_sizes=(1,),
        mode=lax.GatherScatterMode.PROMISE_IN_BOUNDS)


def _bucket_of(v):
    return lax.shift_right_logical(
        lax.shift_right_logical(v, 7) * _MAGIC, 18)


def _impl(dataT, idx):
    mesh = plsc.VectorSubcoreMesh(core_axis_name="c", subcore_axis_name="s")

    @functools.partial(
        pl.kernel,
        mesh=mesh,
        out_type=jax.ShapeDtypeStruct((_B, 2 * _D), jnp.float32),
        scratch_types=[
            pltpu.VMEM((_ICHUNK,), jnp.int32),
            pltpu.VMEM((_ICHUNK,), jnp.int32),
            pltpu.VMEM((_B + _L,), jnp.int32),
            pltpu.VMEM((_B + _L,), jnp.int32),
            pltpu.VMEM((_B + _FLUSH,), jnp.int32),
            pltpu.VMEM((_B + _FLUSH,), jnp.int32),
            pltpu.VMEM((_D, _RANGE), jnp.float32),
            pltpu.VMEM((_D, _RANGE), jnp.float32),
            pltpu.VMEM((_FLUSH, 2 * _D), jnp.float32),
            pltpu.VMEM((_FLUSH, 2 * _D), jnp.float32),
            pltpu.VMEM((_B // _FLUSH + _NPASS + 1, _FLUSH), jnp.int32),
            pltpu.SemaphoreType.DMA,
            pltpu.SemaphoreType.DMA,
            pltpu.SemaphoreType.DMA,
            pltpu.SemaphoreType.DMA,
            pltpu.SemaphoreType.DMA,
        ],
        compiler_params=pltpu.CompilerParams(needs_layout_passes=False),
    )
    def k(tab_hbm, idx_hbm, scr_hbm, idxc0, idxc1, nA, jA, nB, jB, win0,
          win1, src_v, src2_v, jrow_v, sem, sem2, wsem0, wsem1, isem):
        wid = lax.axis_index("s") * _NUM_CORES + lax.axis_index("c")
        iota = lax.iota(jnp.int32, _L)
        wins = (win0, win1)
        wsems = (wsem0, wsem1)
        idxcs = (idxc0, idxc1)

        def wstart(p):
            lo = (wid + p * _NUM_WORKERS) * _RANGE
            ws = pl.multiple_of(jnp.minimum(lo, _WS_LAST), 128)
            return pltpu.async_copy(
                tab_hbm.at[:, pl.ds(ws, _RANGE)], wins[p % 2], wsems[p % 2])

        wcp = {0: wstart(0)}

        icps = [
            pltpu.async_copy(
                idx_hbm.at[pl.ds(c * _ICHUNK, _ICHUNK)], idxcs[c], isem)
            for c in range(_B // _ICHUNK)
        ]
        cntA = jnp.int32(0)
        for c in range(_B // _ICHUNK):
            icps[c].wait()
            idxc_v = idxcs[c]

            def scan_body(g, cnt, c=c, idxc_v=idxc_v):
                cnt0 = cnt
                for u in range(2):
                    gg = g * 2 + u
                    v = idxc_v[pl.ds(gg * _L, _L)]
                    jv = c * _ICHUNK + gg * _L + iota
                    m = (_bucket_of(v) & (_NUM_WORKERS - 1)) == wid
                    plsc.store_compressed(nA.at[pl.ds(cnt0, _L)], v, mask=m)
                    plsc.store_compressed(jA.at[pl.ds(cnt0, _L)], jv, mask=m)
                    cnt0 = cnt0 + jnp.sum(m.astype(jnp.int32))
                return cnt0

            cntA = lax.fori_loop(0, _ICHUNK // _L // 2, scan_body, cntA)
        nA[pl.ds(cntA, _L)] = jnp.full((_L,), _PAD_N, jnp.int32)

        scp_prev = None
        for p in range(_NPASS):
            if p + 1 < _NPASS:
                wcp[p + 1] = wstart(p + 1)

            lo = (wid + p * _NUM_WORKERS) * _RANGE
            ws = jnp.minimum(lo, _WS_LAST)
            win = wins[p % 2]

            def sel_body(g, cnt, p=p):
                v = nA[pl.ds(g * _L, _L)]
                jv = jA[pl.ds(g * _L, _L)]
                m = lax.shift_right_logical(_bucket_of(v), 5) == p
                plsc.store_compressed(nB.at[pl.ds(cnt, _L)], v, mask=m)
                plsc.store_compressed(jB.at[pl.ds(cnt, _L)], jv, mask=m)
                return cnt + jnp.sum(m.astype(jnp.int32))

            ngrp = (cntA + _L) // _L
            cnt = lax.fori_loop(0, ngrp, sel_body, jnp.int32(0))

            for t in range(_FLUSH // _L):
                jB[pl.ds(cnt + t * _L, _L)] = jnp.full((_L,), -1, jnp.int32)
                nB[pl.ds(cnt + t * _L, _L)] = jnp.broadcast_to(ws, (_L,))

            wcp[p].wait()
            if scp_prev is not None:
                scp_prev.wait()

            def ext_row(r, carry, win=win, ws=ws, dst=src_v, base=0):
                i = base + r
                grp = lax.shift_left(lax.shift_right_logical(i, 4), 4)
                nsegv = nB[pl.ds(grp, _L)] - ws
                nn = lax.gather(
                    nsegv,
                    jnp.broadcast_to(
                        lax.bitwise_and(i, _L - 1), (_L, 1)),
                    dimension_numbers=lax.GatherDimensionNumbers(
                        offset_dims=(), collapsed_slice_dims=(0,),
                        start_index_map=(0,)),
                    slice_sizes=(1,),
                    mode=lax.GatherScatterMode.PROMISE_IN_BOUNDS)
                for kk in range(_D // _L):
                    dst[r, pl.ds(kk * _L, _L)] = plsc.load_gather(
                        win, [iota + kk * _L, nn])
                return carry

            for t in range(_FLUSH // _L):
                jrow_v[p, pl.ds(t * _L, _L)] = jB[pl.ds(t * _L, _L)]
            lax.fori_loop(0, _FLUSH, functools.partial(ext_row), jnp.int32(0))
            scp_prev = pltpu.async_copy(
                src_v,
                scr_hbm.at[plsc.Indices(jrow_v.at[p], ignored_value=-1)],
                sem)

            def over_body(f, carry, p=p, win=win, ws=ws):
                for t in range(_FLUSH // _L):
                    jrow_v[_NPASS + f - 1, pl.ds(t * _L, _L)] = (
                        jB[pl.ds(f * _FLUSH + t * _L, _L)])
                lax.fori_loop(
                    0, _FLUSH,
                    functools.partial(ext_row, win=win, ws=ws, dst=src2_v,
                                      base=f * _FLUSH),
                    jnp.int32(0))
                pltpu.async_copy(
                    src2_v,
                    scr_hbm.at[
                        plsc.Indices(jrow_v.at[_NPASS + f - 1],
                                     ignored_value=-1)],
                    sem2).wait()
                return carry

            nflush = (cnt + _FLUSH - 1) // _FLUSH
            lax.fori_loop(1, nflush, over_body, jnp.int32(0))

        scp_prev.wait()

    return k(dataT, idx)


def kernel(data, channelindex):
    dataT = data.T
    idx = channelindex.astype(jnp.int32)
    scr = _impl(dataT, idx)
    return scr[:, :_D]
